# baseline (speedup 1.0000x reference)
import functools

import jax
import jax.numpy as jnp
from jax import lax
from jax.experimental import pallas as pl
from jax.experimental.pallas import tpu as pltpu
from jax.experimental.pallas import tpu_sc as plsc


_CHUNK = 128


def _sc_gather_body(
    nchunks, table_hbm, idx_hbm, out_hbm, idx_v, rows0, rows1, sem0, sem1
):
    nc = 2
    wid = lax.axis_index("s") * nc + lax.axis_index("c")
    b_per_w = nchunks * _CHUNK
    base = wid * b_per_w
    pltpu.sync_copy(idx_hbm.at[pl.ds(base, b_per_w)], idx_v)

    def gstart(j, rows, sem):
        off = pl.multiple_of(j * _CHUNK, _CHUNK)
        pltpu.async_copy(
            table_hbm.at[idx_v.at[pl.ds(off, _CHUNK)]], rows, sem
        )

    def gwait(rows, sem):
        pltpu.make_async_copy(
            table_hbm.at[idx_v.at[pl.ds(0, _CHUNK)]], rows, sem
        ).wait()

    def wb(j, rows):
        off = pl.multiple_of(j * _CHUNK, _CHUNK)
        pltpu.sync_copy(rows, out_hbm.at[pl.ds(base + off, _CHUNK)])

    gstart(0, rows0, sem0)

    def pair(i, carry):
        j0 = 2 * i
        gstart(j0 + 1, rows1, sem1)
        gwait(rows0, sem0)
        wb(j0, rows0)
        gstart(j0 + 2, rows0, sem0)
        gwait(rows1, sem1)
        wb(j0 + 1, rows1)
        return carry

    lax.fori_loop(0, nchunks // 2 - 1, pair, 0)
    j0 = nchunks - 2
    gstart(j0 + 1, rows1, sem1)
    gwait(rows0, sem0)
    wb(j0, rows0)
    gwait(rows1, sem1)
    wb(j0 + 1, rows1)


def _sc_gather(tpack, gids):
    ng, dw = tpack.shape
    (b,) = gids.shape
    nw = 32
    assert b % (nw * _CHUNK) == 0 and (b // (nw * _CHUNK)) % 2 == 0
    nchunks = b // (nw * _CHUNK)
    mesh = plsc.VectorSubcoreMesh(core_axis_name="c", subcore_axis_name="s")
    kern = functools.partial(
        pl.kernel,
        mesh=mesh,
        out_type=jax.ShapeDtypeStruct((b, dw), tpack.dtype),
        scratch_types=[
            pltpu.VMEM((nchunks * _CHUNK,), jnp.int32),
            pltpu.VMEM((_CHUNK, dw), tpack.dtype),
            pltpu.VMEM((_CHUNK, dw), tpack.dtype),
            pltpu.SemaphoreType.DMA,
            pltpu.SemaphoreType.DMA,
        ],
    )(functools.partial(_sc_gather_body, nchunks))
    return kern(tpack, gids)


def _proj_body(ids_ref, y_ref, p_ref, o_ref):
    e = p_ref.shape[0]
    he = e // 2
    bm = y_ref.shape[0]
    ids = ids_ref[0, 0, :]
    mh = ((ids >> 1) & 1).reshape(bm, 1) == 1
    msw = (ids & 1).reshape(bm, 1) == 1
    y = y_ref[...]
    w2 = jnp.where(mh, y[:, e:], y[:, :e])
    wq = jnp.where(msw, w2[:, he:], w2[:, :he])
    lo = jax.lax.bitcast_convert_type(wq << 16, jnp.float32)
    hi = jax.lax.bitcast_convert_type(wq & jnp.uint32(0xFFFF0000), jnp.float32)
    x = jnp.concatenate([lo, hi], axis=1)
    o_ref[...] = jnp.dot(x, p_ref[...], preferred_element_type=jnp.float32)


def _tc_project(y, ids, p):
    r = y.shape[0]
    e, h = p.shape
    bm = 2048
    assert r % bm == 0
    ids3 = ids.reshape(r // bm, 1, bm)
    return pl.pallas_call(
        _proj_body,
        grid=(r // bm,),
        in_specs=[
            pl.BlockSpec((1, 1, bm), lambda i: (i, 0, 0)),
            pl.BlockSpec((bm, 2 * e), lambda i: (i, 0)),
            pl.BlockSpec((e, h), lambda i: (0, 0)),
        ],
        out_specs=pl.BlockSpec((bm, h), lambda i: (i, 0)),
        out_shape=jax.ShapeDtypeStruct((r, h), jnp.float32),
    )(ids3, y, p)


_BW = 2048


def _pack_body(tt_ref, o_ref):
    e = tt_ref.shape[0]
    bw = tt_ref.shape[1]
    x = tt_ref[...]
    i1 = lax.broadcasted_iota(jnp.int32, (e, e), 0)
    i2 = lax.broadcasted_iota(jnp.int32, (e, e), 1)
    eye = (i1 == i2).astype(jnp.float32)
    xt = jax.lax.dot_general(
        x, eye, (((0,), (0,)), ((), ())), preferred_element_type=jnp.float32
    )
    u = jax.lax.bitcast_convert_type(xt, jnp.uint32)
    rnd = u + jnp.uint32(0x7FFF) + ((u >> 16) & jnp.uint32(1))
    he = e // 2
    o_ref[...] = (rnd[:, :he] >> 16) | (rnd[:, he:] & jnp.uint32(0xFFFF0000))


def _pack_table(table):
    v, e = table.shape
    tt = table.T
    grid = (v + _BW - 1) // _BW
    packed = pl.pallas_call(
        _pack_body,
        grid=(grid,),
        in_specs=[pl.BlockSpec((e, _BW), lambda i: (0, i))],
        out_specs=pl.BlockSpec((_BW, e // 2), lambda i: (i, 0)),
        out_shape=jax.ShapeDtypeStruct((v, e // 2), jnp.uint32),
    )(tt)
    return packed.reshape(v // 4, 2 * e)


def kernel(inputs, weight_embedding_table, project_variable):
    batch, seq = inputs.shape
    h = project_variable.shape[1]
    ids_t = inputs.T.reshape(-1).astype(jnp.int32)
    tpack = _pack_table(weight_embedding_table)
    y = _sc_gather(tpack, ids_t >> 2)
    outt = _tc_project(y, ids_t, project_variable)
    return outt.reshape(seq, batch, h).transpose(1, 0, 2)


# speedup vs baseline: 2.2672x; 2.2672x over previous
import functools

import jax
import jax.numpy as jnp
from jax import lax
from jax.experimental import pallas as pl
from jax.experimental.pallas import tpu as pltpu
from jax.experimental.pallas import tpu_sc as plsc


_CHUNK = 128


def _sc_gather_body(
    nchunks, table_hbm, idx_hbm, out_hbm, idx_v, rows0, rows1, sem0, sem1
):
    nc = 2
    wid = lax.axis_index("s") * nc + lax.axis_index("c")
    b_per_w = nchunks * _CHUNK
    base = wid * b_per_w
    pltpu.sync_copy(idx_hbm.at[pl.ds(base, b_per_w)], idx_v)

    def gstart(j, rows, sem):
        off = pl.multiple_of(j * _CHUNK, _CHUNK)
        pltpu.async_copy(
            table_hbm.at[idx_v.at[pl.ds(off, _CHUNK)]], rows, sem
        )

    def gwait(rows, sem):
        pltpu.make_async_copy(
            table_hbm.at[idx_v.at[pl.ds(0, _CHUNK)]], rows, sem
        ).wait()

    def wb(j, rows):
        off = pl.multiple_of(j * _CHUNK, _CHUNK)
        pltpu.sync_copy(rows, out_hbm.at[pl.ds(base + off, _CHUNK)])

    gstart(0, rows0, sem0)

    def pair(i, carry):
        j0 = 2 * i
        gstart(j0 + 1, rows1, sem1)
        gwait(rows0, sem0)
        wb(j0, rows0)
        gstart(j0 + 2, rows0, sem0)
        gwait(rows1, sem1)
        wb(j0 + 1, rows1)
        return carry

    lax.fori_loop(0, nchunks // 2 - 1, pair, 0)
    j0 = nchunks - 2
    gstart(j0 + 1, rows1, sem1)
    gwait(rows0, sem0)
    wb(j0, rows0)
    gwait(rows1, sem1)
    wb(j0 + 1, rows1)


def _sc_gather(tpack, gids):
    ng, dw = tpack.shape
    (b,) = gids.shape
    nw = 32
    assert b % (nw * _CHUNK) == 0 and (b // (nw * _CHUNK)) % 2 == 0
    nchunks = b // (nw * _CHUNK)
    mesh = plsc.VectorSubcoreMesh(core_axis_name="c", subcore_axis_name="s")
    kern = functools.partial(
        pl.kernel,
        mesh=mesh,
        out_type=jax.ShapeDtypeStruct((b, dw), tpack.dtype),
        scratch_types=[
            pltpu.VMEM((nchunks * _CHUNK,), jnp.int32),
            pltpu.VMEM((_CHUNK, dw), tpack.dtype),
            pltpu.VMEM((_CHUNK, dw), tpack.dtype),
            pltpu.SemaphoreType.DMA,
            pltpu.SemaphoreType.DMA,
        ],
    )(functools.partial(_sc_gather_body, nchunks))
    return kern(tpack, gids)


def _proj_body(ids_ref, y_ref, p_ref, o_ref):
    e = p_ref.shape[0]
    bm = y_ref.shape[0]
    ids = ids_ref[0, 0, :]
    par = ((ids >> 7) & 1).astype(jnp.float32).reshape(bm, 1)
    y = y_ref[...]
    a = y[:, :e]
    b = y[:, e:]
    x = a + (b - a) * par
    o_ref[...] = jnp.dot(x, p_ref[...], preferred_element_type=jnp.float32)


def _tc_project(y, ids, p):
    r = y.shape[0]
    e, h = p.shape
    bm = 2048
    assert r % bm == 0
    ids3 = ids.reshape(r // bm, 1, bm)
    return pl.pallas_call(
        _proj_body,
        grid=(r // bm,),
        in_specs=[
            pl.BlockSpec((1, 1, bm), lambda i: (i, 0, 0)),
            pl.BlockSpec((bm, 2 * e), lambda i: (i, 0)),
            pl.BlockSpec((e, h), lambda i: (0, 0)),
        ],
        out_specs=pl.BlockSpec((bm, h), lambda i: (i, 0)),
        out_shape=jax.ShapeDtypeStruct((r, h), jnp.float32),
    )(ids3, y, p)


_BW = 8192


def _pack_body(tt_ref, o_ref):
    e = tt_ref.shape[0]
    bw = tt_ref.shape[1]
    x = tt_ref[...]
    chunks = []
    for t in range(bw // (4 * e)):
        a = x[:, 4 * e * t : 4 * e * t + 2 * e]
        b = x[:, 4 * e * t + 2 * e : 4 * e * t + 4 * e]
        chunks.append(jnp.concatenate([a.T, b.T], axis=1))
    o_ref[...] = jnp.concatenate(chunks, axis=0)


def _pack_table(table):
    v, e = table.shape
    tt = table.T
    grid = (v + _BW - 1) // _BW
    return pl.pallas_call(
        _pack_body,
        grid=(grid,),
        in_specs=[pl.BlockSpec((e, _BW), lambda i: (0, i))],
        out_specs=pl.BlockSpec((_BW // 2, 2 * e), lambda i: (i, 0)),
        out_shape=jax.ShapeDtypeStruct(((v + 1) // 2, 2 * e), jnp.float32),
    )(tt)


def kernel(inputs, weight_embedding_table, project_variable):
    batch, seq = inputs.shape
    h = project_variable.shape[1]
    ids_t = inputs.T.reshape(-1).astype(jnp.int32)
    tpack = _pack_table(weight_embedding_table)
    gids = (
        (ids_t >> 13) * (_BW // 2)
        + ((ids_t >> 8) & (_BW // 256 - 1)) * 128
        + (ids_t & 127)
    )
    y = _sc_gather(tpack, gids)
    outt = _tc_project(y, ids_t, project_variable)
    return outt.reshape(seq, batch, h).transpose(1, 0, 2)
